# Initial kernel scaffold; baseline (speedup 1.0000x reference)
import jax
import jax.numpy as jnp
from jax.experimental import pallas as pl


def kernel(x, W, u, v, widx, targets):
    raise NotImplementedError("write your pallas kernel here")


# speedup vs baseline: 6.3671x; 6.3671x over previous
import functools

import jax
import jax.numpy as jnp
from jax import lax
from jax.experimental import pallas as pl
from jax.experimental.pallas import tpu as pltpu
from jax.experimental.pallas import tpu_sc as plsc

N_NODES = 10000
D = 128
N_WEIGHTS = 1024
N_EDGES = 320000
N_TARGETS = 2000

NC = 2
NS = 16
NW = NC * NS
EB = 128
NB = 80
E_PAD = NW * NB * EB
EPW = NB * EB
STRIPE = 640
ROWS_PAD = NS * STRIPE
W_PAD = N_WEIGHTS + 16
T_PAD = 2048


def _sc_body(x_hbm, w_hbm, u_hbm, v_hbm, widx_hbm, tgt_hbm, out_hbm,
             acc, u_loc, v_loc, widx_loc, w_loc, rows, tgt_loc, sem):
    c = lax.axis_index("c")
    s = lax.axis_index("s")
    wid = c * NS + s
    stripe = s * STRIPE

    pltpu.sync_copy(u_hbm.at[pl.ds(wid * NB, NB)], u_loc)
    pltpu.sync_copy(v_hbm.at[pl.ds(wid * NB, NB)], v_loc)
    pltpu.sync_copy(widx_hbm.at[pl.ds(wid * EPW, EPW)], widx_loc)
    pltpu.sync_copy(w_hbm, w_loc)

    zv = jnp.zeros((16,), jnp.float32)

    def _zero_row(i, carry):
        for f in range(D // 16):
            rows[i, pl.ds(f * 16, 16)] = zv
        return carry

    lax.fori_loop(0, EB, _zero_row, 0)

    @pl.when(c == 0)
    def _():
        pltpu.sync_copy(x_hbm.at[pl.ds(stripe, STRIPE)],
                        acc.at[pl.ds(stripe, STRIPE)])

    @pl.when(c != 0)
    def _():
        for k in range(STRIPE // EB):
            pltpu.sync_copy(rows, acc.at[pl.ds(stripe + k * EB, EB)])

    plsc.subcore_barrier()

    @pl.when(c == 0)
    def _():
        pltpu.sync_copy(tgt_hbm.at[pl.ds(s * EB, EB)], tgt_loc)
        pltpu.sync_copy(rows, acc.at[tgt_loc])

    plsc.subcore_barrier()

    def _batch(b, carry):
        pltpu.async_copy(x_hbm.at[u_loc.at[b]], rows, sem).wait()

        def _group(g, carry2):
            wi = widx_loc[pl.ds(b * EB + g * 16, 16)]
            wv16 = plsc.load_gather(w_loc, [wi])
            for j in range(16):
                wj = jnp.full((16,), wv16[j], jnp.float32)
                e = g * 16 + j
                for f in range(D // 16):
                    rows[e, pl.ds(f * 16, 16)] = (
                        rows[e, pl.ds(f * 16, 16)] * wj)
            return carry2

        lax.fori_loop(0, EB // 16, _group, 0)

        pltpu.sync_copy(rows, acc.at[v_loc.at[b]], add=True)
        return carry

    lax.fori_loop(0, NB, _batch, 0)

    plsc.subcore_barrier()

    pltpu.sync_copy(acc.at[pl.ds(stripe, STRIPE)],
                    out_hbm.at[c, pl.ds(stripe, STRIPE)])


_sc_call = pl.kernel(
    _sc_body,
    out_type=jax.ShapeDtypeStruct((NC, ROWS_PAD, D), jnp.float32),
    mesh=plsc.VectorSubcoreMesh(
        core_axis_name="c", subcore_axis_name="s",
        num_cores=NC, num_subcores=NS),
    compiler_params=pltpu.CompilerParams(needs_layout_passes=False),
    scratch_types=[
        pltpu.VMEM_SHARED((ROWS_PAD, D), jnp.float32),
        pltpu.VMEM((NB, EB), jnp.int32),
        pltpu.VMEM((NB, EB), jnp.int32),
        pltpu.VMEM((EPW,), jnp.int32),
        pltpu.VMEM((W_PAD,), jnp.float32),
        pltpu.VMEM((EB, D), jnp.float32),
        pltpu.VMEM((EB,), jnp.int32),
        pltpu.SemaphoreType.DMA,
    ],
)


def _combine_body(p_ref, o_ref):
    o_ref[...] = p_ref[0] + p_ref[1]


_combine = pl.pallas_call(
    _combine_body,
    grid=(10,),
    in_specs=[pl.BlockSpec((NC, 1000, D), lambda i: (0, i, 0))],
    out_specs=pl.BlockSpec((1000, D), lambda i: (i, 0)),
    out_shape=jax.ShapeDtypeStruct((N_NODES, D), jnp.float32),
)


def kernel(x, W, u, v, widx, targets):
    u = u.astype(jnp.int32)
    v = v.astype(jnp.int32)
    widx = widx.astype(jnp.int32)
    targets = targets.astype(jnp.int32)

    x_pad = jnp.concatenate(
        [x, jnp.zeros((ROWS_PAD - N_NODES, D), x.dtype)], axis=0)
    w_pad = jnp.concatenate([W, jnp.zeros((W_PAD - N_WEIGHTS,), W.dtype)])
    pad_e = E_PAD - N_EDGES
    u_p = jnp.concatenate([u, jnp.zeros((pad_e,), jnp.int32)]).reshape(
        NW * NB, EB)
    v_p = jnp.concatenate([v, jnp.zeros((pad_e,), jnp.int32)]).reshape(
        NW * NB, EB)
    widx_p = jnp.concatenate(
        [widx, jnp.full((pad_e,), N_WEIGHTS, jnp.int32)])
    tgt_p = jnp.concatenate(
        [targets, jnp.full((T_PAD - N_TARGETS,), N_NODES, jnp.int32)])

    partials = _sc_call(x_pad, w_pad, u_p, v_p, widx_p, tgt_p)
    return _combine(partials)
